# Initial kernel scaffold; baseline (speedup 1.0000x reference)
import jax
import jax.numpy as jnp
from jax.experimental import pallas as pl


def kernel(t):
    raise NotImplementedError("write your pallas kernel here")


# speedup vs baseline: 3.3628x; 3.3628x over previous
import functools

import jax
import jax.numpy as jnp
from jax.experimental import pallas as pl
from jax.experimental.pallas import tpu as pltpu

N = 1024
S_B = 32
LCHUNK = 128
SCHUNK = 8


def _body(a_ref, b_ref, tp_ref, o_ref):
    acc0 = jnp.full((S_B, N), -jnp.inf, dtype=jnp.float32)

    def cstep(mc, acc):
        ac = a_ref[0, :, pl.ds(pl.multiple_of(mc * LCHUNK, LCHUNK), LCHUNK)]

        def bstep(mb, acc):
            base = pl.multiple_of(mc * LCHUNK + mb * SCHUNK, SCHUNK)
            bc = b_ref[0, pl.ds(base, SCHUNK), :]
            for j in range(SCHUNK):
                jj = mb * SCHUNK + j
                idx = jnp.full((S_B, LCHUNK), jj, dtype=jnp.int32)
                a_bc = jnp.take_along_axis(ac, idx, axis=1)
                a_col = a_bc[:, :1]
                b_row = bc[j : j + 1, :]
                acc = jnp.maximum(acc, jnp.minimum(a_col, b_row))
            return acc

        return jax.lax.fori_loop(0, LCHUNK // SCHUNK, bstep, acc)

    acc = jax.lax.fori_loop(0, N // LCHUNK, cstep, acc0)
    tp = tp_ref[0]
    o_ref[0] = tp + acc - tp * acc


@jax.jit
def kernel(t):
    grid = (2, N // S_B)
    return pl.pallas_call(
        _body,
        grid=grid,
        in_specs=[
            pl.BlockSpec((1, S_B, N), lambda p, si: (0, si, 0)),
            pl.BlockSpec((1, N, N), lambda p, si: (p, 0, 0)),
            pl.BlockSpec((1, S_B, N), lambda p, si: (p, si, 0)),
        ],
        out_specs=pl.BlockSpec((1, S_B, N), lambda p, si: (p, si, 0)),
        out_shape=jax.ShapeDtypeStruct((2, N, N), jnp.float32),
        compiler_params=pltpu.CompilerParams(
            dimension_semantics=("arbitrary", "arbitrary"),
        ),
    )(t, t, t)
